# baseline (speedup 1.0000x reference)
import functools

import jax
import jax.numpy as jnp
from jax import lax
from jax.experimental import pallas as pl
from jax.experimental.pallas import tpu as pltpu
from jax.experimental.pallas import tpu_sc as plsc

EMBED_DIM = 64
POS_DIM = 32
OUT_DIM = EMBED_DIM + 2 * POS_DIM
B, L = 4096, 200

NC, NS = 2, 16
NW = NC * NS
ROWS_PER_W = B // NW
STEPS = ROWS_PER_W
NBUF = 4
SPLITS = ((0, 128), (128, 72))


def _emb_body(wid_hbm, p1_hbm, p2_hbm, wtab_hbm, ptab_hbm, out_hbm,
              widx, p1idx, p2idx, wbuf, p1buf, p2buf,
              isem, gsem, wsem):
    c = lax.axis_index("c")
    s = lax.axis_index("s")
    wid = s * NC + c
    row0 = wid * ROWS_PER_W

    def idx_copies(i, b):
        bi = row0 + i
        return [
            pltpu.make_async_copy(wid_hbm.at[bi], widx.at[b], isem.at[b]),
            pltpu.make_async_copy(p1_hbm.at[bi], p1idx.at[b], isem.at[b]),
            pltpu.make_async_copy(p2_hbm.at[bi], p2idx.at[b], isem.at[b]),
        ]

    def gather_copies(b):
        out = []
        for off, n in SPLITS:
            sl = pl.ds(off, n)
            out.append(pltpu.make_async_copy(
                wtab_hbm.at[widx.at[b, sl]], wbuf.at[b, sl], gsem.at[b]))
            out.append(pltpu.make_async_copy(
                ptab_hbm.at[p1idx.at[b, sl]], p1buf.at[b, sl], gsem.at[b]))
            out.append(pltpu.make_async_copy(
                ptab_hbm.at[p2idx.at[b, sl]], p2buf.at[b, sl], gsem.at[b]))
        return out

    def write_copies(i, b):
        bi = row0 + i
        return [
            pltpu.make_async_copy(
                wbuf.at[b], out_hbm.at[bi, slice(None), pl.ds(0, EMBED_DIM)],
                wsem.at[b]),
            pltpu.make_async_copy(
                p1buf.at[b],
                out_hbm.at[bi, slice(None), pl.ds(EMBED_DIM, POS_DIM)],
                wsem.at[b]),
            pltpu.make_async_copy(
                p2buf.at[b],
                out_hbm.at[bi, slice(None), pl.ds(EMBED_DIM + POS_DIM, POS_DIM)],
                wsem.at[b]),
        ]

    def start(copies):
        for cp in copies:
            cp.start()

    def wait(copies):
        for cp in copies:
            cp.wait()

    start(idx_copies(0, 0))
    start(idx_copies(1, 1))
    wait(idx_copies(0, 0))
    start(gather_copies(0))
    wait(idx_copies(1, 1))
    start(gather_copies(1))
    start(idx_copies(2, 2))

    def phase(i, p):
        @pl.when(i >= 2)
        def _():
            wait(write_copies(i - 2, (p + 2) % NBUF))

        @pl.when(i + 2 < STEPS)
        def _():
            wait(idx_copies(i + 2, (p + 2) % NBUF))
            start(gather_copies((p + 2) % NBUF))

        wait(gather_copies(p))
        start(write_copies(i, p))

        @pl.when(i + 3 < STEPS)
        def _():
            start(idx_copies(i + 3, (p + 3) % NBUF))

    def body(j, carry):
        i0 = NBUF * j
        for p in range(NBUF):
            phase(i0 + p, p)
        return carry

    lax.fori_loop(0, STEPS // NBUF, body, 0)

    wait(write_copies(STEPS - 2, (STEPS - 2) % NBUF))
    wait(write_copies(STEPS - 1, (STEPS - 1) % NBUF))


@functools.partial(
    pl.kernel,
    out_type=jax.ShapeDtypeStruct((B, L, OUT_DIM), jnp.float32),
    mesh=plsc.VectorSubcoreMesh(core_axis_name="c", subcore_axis_name="s"),
    compiler_params=pltpu.CompilerParams(use_tc_tiling_on_sc=False),
    scratch_types=[
        pltpu.VMEM((NBUF, L), jnp.int32),
        pltpu.VMEM((NBUF, L), jnp.int32),
        pltpu.VMEM((NBUF, L), jnp.int32),
        pltpu.VMEM((NBUF, L, EMBED_DIM), jnp.float32),
        pltpu.VMEM((NBUF, L, POS_DIM), jnp.float32),
        pltpu.VMEM((NBUF, L, POS_DIM), jnp.float32),
        pltpu.SemaphoreType.DMA((NBUF,)),
        pltpu.SemaphoreType.DMA((NBUF,)),
        pltpu.SemaphoreType.DMA((NBUF,)),
    ],
)
def _emb_kernel(*refs):
    _emb_body(*refs)


def kernel(word_id, pos_1, pos_2, word_table, pos_table):
    return _emb_kernel(word_id, pos_1, pos_2, word_table, pos_table)


# speedup vs baseline: 1.6911x; 1.6911x over previous
import functools

import jax
import jax.numpy as jnp
from jax import lax
from jax.experimental import pallas as pl
from jax.experimental.pallas import tpu as pltpu
from jax.experimental.pallas import tpu_sc as plsc

EMBED_DIM = 64
POS_DIM = 32
POS_VOCAB = 201
OUT_DIM = EMBED_DIM + 2 * POS_DIM
B, L = 4096, 200
LPAD = 208

NC, NS = 2, 16
NW = NC * NS
ROWS_PER_W = B // NW
STEPS = ROWS_PER_W
NBUF = 4
SPLITS = ((0, 128), (128, 72))


def _emb_body(wid_hbm, p1_hbm, p2_hbm, wtab_hbm, ptab2_hbm, out_hbm,
              widx, p1idx, p2idx, pidx, wbuf, pbuf,
              isem, gsem, wsem):
    c = lax.axis_index("c")
    s = lax.axis_index("s")
    wid = s * NC + c
    row0 = wid * ROWS_PER_W

    def idx_copies(i, b):
        bi = row0 + i
        return [
            pltpu.make_async_copy(wid_hbm.at[bi], widx.at[b, pl.ds(0, L)],
                                  isem.at[b]),
            pltpu.make_async_copy(p1_hbm.at[bi], p1idx.at[b, pl.ds(0, L)],
                                  isem.at[b]),
            pltpu.make_async_copy(p2_hbm.at[bi], p2idx.at[b, pl.ds(0, L)],
                                  isem.at[b]),
        ]

    def pair_indices(b):
        for k in range(LPAD // 16):
            sl = pl.ds(k * 16, 16)
            pidx[b, sl] = p1idx[b, sl] * POS_VOCAB + p2idx[b, sl]

    def gather_copies(b):
        out = []
        for off, n in SPLITS:
            sl = pl.ds(off, n)
            out.append(pltpu.make_async_copy(
                wtab_hbm.at[widx.at[b, sl]], wbuf.at[b, sl], gsem.at[b]))
            out.append(pltpu.make_async_copy(
                ptab2_hbm.at[pidx.at[b, sl]], pbuf.at[b, sl], gsem.at[b]))
        return out

    def write_copies(i, b):
        bi = row0 + i
        return [
            pltpu.make_async_copy(
                wbuf.at[b], out_hbm.at[bi, slice(None), pl.ds(0, EMBED_DIM)],
                wsem.at[b]),
            pltpu.make_async_copy(
                pbuf.at[b],
                out_hbm.at[bi, slice(None), pl.ds(EMBED_DIM, 2 * POS_DIM)],
                wsem.at[b]),
        ]

    def start(copies):
        for cp in copies:
            cp.start()

    def wait(copies):
        for cp in copies:
            cp.wait()

    start(idx_copies(0, 0))
    start(idx_copies(1, 1))
    wait(idx_copies(0, 0))
    pair_indices(0)
    start(gather_copies(0))
    wait(idx_copies(1, 1))
    pair_indices(1)
    start(gather_copies(1))
    start(idx_copies(2, 2))

    def phase(i, p):
        @pl.when(i >= 2)
        def _():
            wait(write_copies(i - 2, (p + 2) % NBUF))

        @pl.when(i + 2 < STEPS)
        def _():
            wait(idx_copies(i + 2, (p + 2) % NBUF))
            pair_indices((p + 2) % NBUF)
            start(gather_copies((p + 2) % NBUF))

        wait(gather_copies(p))
        start(write_copies(i, p))

        @pl.when(i + 3 < STEPS)
        def _():
            start(idx_copies(i + 3, (p + 3) % NBUF))

    def body(j, carry):
        i0 = NBUF * j
        for p in range(NBUF):
            phase(i0 + p, p)
        return carry

    lax.fori_loop(0, STEPS // NBUF, body, 0)

    wait(write_copies(STEPS - 2, (STEPS - 2) % NBUF))
    wait(write_copies(STEPS - 1, (STEPS - 1) % NBUF))


@functools.partial(
    pl.kernel,
    out_type=jax.ShapeDtypeStruct((B, L, OUT_DIM), jnp.float32),
    mesh=plsc.VectorSubcoreMesh(core_axis_name="c", subcore_axis_name="s"),
    compiler_params=pltpu.CompilerParams(use_tc_tiling_on_sc=False),
    scratch_types=[
        pltpu.VMEM((NBUF, LPAD), jnp.int32),
        pltpu.VMEM((NBUF, LPAD), jnp.int32),
        pltpu.VMEM((NBUF, LPAD), jnp.int32),
        pltpu.VMEM((NBUF, LPAD), jnp.int32),
        pltpu.VMEM((NBUF, L, EMBED_DIM), jnp.float32),
        pltpu.VMEM((NBUF, L, 2 * POS_DIM), jnp.float32),
        pltpu.SemaphoreType.DMA((NBUF,)),
        pltpu.SemaphoreType.DMA((NBUF,)),
        pltpu.SemaphoreType.DMA((NBUF,)),
    ],
)
def _emb_kernel(*refs):
    _emb_body(*refs)


def kernel(word_id, pos_1, pos_2, word_table, pos_table):
    pair_table = jnp.concatenate(
        [
            jnp.broadcast_to(pos_table[:, None, :],
                             (POS_VOCAB, POS_VOCAB, POS_DIM)),
            jnp.broadcast_to(pos_table[None, :, :],
                             (POS_VOCAB, POS_VOCAB, POS_DIM)),
        ],
        axis=-1,
    ).reshape(POS_VOCAB * POS_VOCAB, EMBED_DIM)
    return _emb_kernel(word_id, pos_1, pos_2, word_table, pair_table)


# speedup vs baseline: 1.6961x; 1.0029x over previous
import functools

import jax
import jax.numpy as jnp
from jax import lax
from jax.experimental import pallas as pl
from jax.experimental.pallas import tpu as pltpu
from jax.experimental.pallas import tpu_sc as plsc

EMBED_DIM = 64
POS_DIM = 32
POS_VOCAB = 201
OUT_DIM = EMBED_DIM + 2 * POS_DIM
B, L = 4096, 200
LPAD = 208

NC, NS = 2, 16
NW = NC * NS
ROWS_PER_W = B // NW
STEPS = ROWS_PER_W
NBUF = 4
SPLITS = ((0, 128), (128, 72))


def _emb_body(wid_hbm, p1_hbm, p2_hbm, wtab_hbm, ptab2_hbm, out_hbm,
              widx, p1idx, p2idx, pidx, wbuf, pbuf,
              isem, gsem, wsem):
    c = lax.axis_index("c")
    s = lax.axis_index("s")
    wid = s * NC + c
    row0 = wid * ROWS_PER_W

    def idx_copies(i, b):
        bi = row0 + i
        return [
            pltpu.make_async_copy(wid_hbm.at[bi], widx.at[b, pl.ds(0, L)],
                                  isem.at[b]),
            pltpu.make_async_copy(p1_hbm.at[bi], p1idx.at[b, pl.ds(0, L)],
                                  isem.at[b]),
            pltpu.make_async_copy(p2_hbm.at[bi], p2idx.at[b, pl.ds(0, L)],
                                  isem.at[b]),
        ]

    def pair_indices(b):
        for k in range(LPAD // 16):
            sl = pl.ds(k * 16, 16)
            pidx[b, sl] = p1idx[b, sl] * POS_VOCAB + p2idx[b, sl]

    def gather_copies(b):
        out = []
        for off, n in SPLITS:
            sl = pl.ds(off, n)
            out.append(pltpu.make_async_copy(
                wtab_hbm.at[widx.at[b, sl]], wbuf.at[b, sl], gsem.at[b]))
            out.append(pltpu.make_async_copy(
                ptab2_hbm.at[pidx.at[b, sl]], pbuf.at[b, sl], gsem.at[b]))
        return out

    def write_copies(i, b):
        bi = row0 + i
        return [
            pltpu.make_async_copy(
                wbuf.at[b], out_hbm.at[bi, slice(None), pl.ds(0, EMBED_DIM)],
                wsem.at[b]),
            pltpu.make_async_copy(
                pbuf.at[b],
                out_hbm.at[bi, slice(None), pl.ds(EMBED_DIM, 2 * POS_DIM)],
                wsem.at[b]),
        ]

    def start(copies):
        for cp in copies:
            cp.start()

    def wait(copies):
        for cp in copies:
            cp.wait()

    start(idx_copies(0, 0))
    start(idx_copies(1, 1))
    wait(idx_copies(0, 0))
    pair_indices(0)
    start(gather_copies(0))
    wait(idx_copies(1, 1))
    pair_indices(1)
    start(gather_copies(1))
    start(idx_copies(2, 2))

    def phase(i, p):
        @pl.when(i >= 2)
        def _():
            wait(write_copies(i - 2, (p + 2) % NBUF))

        @pl.when(i + 2 < STEPS)
        def _():
            wait(idx_copies(i + 2, (p + 2) % NBUF))
            pair_indices((p + 2) % NBUF)
            start(gather_copies((p + 2) % NBUF))

        wait(gather_copies(p))
        start(write_copies(i, p))

        @pl.when(i + 3 < STEPS)
        def _():
            start(idx_copies(i + 3, (p + 3) % NBUF))

    def body(j, carry):
        i0 = NBUF * j
        for p in range(NBUF):
            phase(i0 + p, p)
        return carry

    lax.fori_loop(0, STEPS // NBUF, body, 0)

    wait(write_copies(STEPS - 2, (STEPS - 2) % NBUF))
    wait(write_copies(STEPS - 1, (STEPS - 1) % NBUF))


@functools.partial(
    pl.kernel,
    out_type=jax.ShapeDtypeStruct((B, L, OUT_DIM), jnp.float32),
    mesh=plsc.VectorSubcoreMesh(core_axis_name="c", subcore_axis_name="s"),
    compiler_params=pltpu.CompilerParams(use_tc_tiling_on_sc=False),
    scratch_types=[
        pltpu.VMEM((NBUF, LPAD), jnp.int32),
        pltpu.VMEM((NBUF, LPAD), jnp.int32),
        pltpu.VMEM((NBUF, LPAD), jnp.int32),
        pltpu.VMEM((NBUF, LPAD), jnp.int32),
        pltpu.VMEM((NBUF, L, EMBED_DIM), jnp.float32),
        pltpu.VMEM((NBUF, L, 2 * POS_DIM), jnp.float32),
        pltpu.SemaphoreType.DMA((NBUF,)),
        pltpu.SemaphoreType.DMA((NBUF,)),
        pltpu.SemaphoreType.DMA((NBUF,)),
    ],
)
def _emb_kernel(*refs):
    _emb_body(*refs)


def kernel(word_id, pos_1, pos_2, word_table, pos_table):
    pair_table = jnp.concatenate(
        [
            jnp.repeat(pos_table, POS_VOCAB, axis=0),
            jnp.tile(pos_table, (POS_VOCAB, 1)),
        ],
        axis=1,
    )
    return _emb_kernel(word_id, pos_1, pos_2, word_table, pair_table)
